# Initial kernel scaffold; baseline (speedup 1.0000x reference)
import jax
import jax.numpy as jnp
from jax.experimental import pallas as pl


def kernel(users, adj_matrix, top_k_indices, top_k_values, temperature):
    raise NotImplementedError("write your pallas kernel here")


# speedup vs baseline: 2.9804x; 2.9804x over previous
import functools

import jax
import jax.numpy as jnp
from jax import lax
from jax.experimental import pallas as pl
from jax.experimental.pallas import tpu as pltpu
from jax.experimental.pallas import tpu_sc as plsc

I = 5000
K = 50
B = 1024
NC = 2
NS = 16
NW = NC * NS
IP = 5120
JP = 5120
KP = 64
RPT = IP // NW
G = 8
NB = RPT // G


def _rsqrt16(x):
  i = plsc.bitcast(x, jnp.int32)
  i = jnp.int32(0x5F3759DF) - lax.shift_right_logical(i, 1)
  y = plsc.bitcast(i, jnp.float32)
  hx = 0.5 * x
  for _ in range(3):
    y = y * (1.5 - hx * y * y)
  return y


def _attn_stats_body(t_ref, v_ref, ew_ref, s_ref):
  ew = jnp.exp(v_ref[...] / t_ref[0])
  ew_ref[...] = ew
  s_ref[...] = jnp.sum(ew, axis=1, keepdims=True)


def _attn_stats(top_k_values, temperature):
  return pl.pallas_call(
      _attn_stats_body,
      in_specs=[
          pl.BlockSpec(memory_space=pltpu.SMEM),
          pl.BlockSpec((I, K), lambda: (0, 0)),
      ],
      out_specs=[
          pl.BlockSpec((I, K), lambda: (0, 0)),
          pl.BlockSpec((I, 1), lambda: (0, 0)),
      ],
      out_shape=[
          jax.ShapeDtypeStruct((I, K), jnp.float32),
          jax.ShapeDtypeStruct((I, 1), jnp.float32),
      ],
  )(temperature, top_k_values)


_IOTA16 = None


def _gather16(x, pos):
  dnums = lax.GatherDimensionNumbers(
      offset_dims=(), collapsed_slice_dims=(0,), start_index_map=(0,))
  return lax.gather(x, pos[:, None], dnums, (1,),
                    mode=lax.GatherScatterMode.PROMISE_IN_BOUNDS)


def _densify_body(idx_hbm, ew_hbm, s_hbm, m_hbm, s_v, idx_v, ew_v, rowg_v):
  cid = lax.axis_index("c")
  sid = lax.axis_index("s")
  wid = sid * NC + cid
  base = wid * RPT

  pltpu.sync_copy(s_hbm, s_v)
  pltpu.sync_copy(idx_hbm.at[pl.ds(base * KP, RPT * KP)], idx_v)
  pltpu.sync_copy(ew_hbm.at[pl.ds(base * KP, RPT * KP)], ew_v)

  zero16f = jnp.zeros((16,), jnp.float32)
  iota = lax.iota(jnp.int32, 16)

  def _zrow(t, c):
    rowg_v[pl.ds(t * 16, 16)] = zero16f
    return c
  lax.fori_loop(0, G * JP // 16, _zrow, 0)

  def _batch(b, c):
    row0 = b * G

    def _build(t, c2):
      g = t // (KP // 16)
      ck = t % (KP // 16)
      r = row0 + g
      off = r * KP + ck * 16
      iv = idx_v[pl.ds(off, 16)]
      ev = ew_v[pl.ds(off, 16)]
      nbr = plsc.load_gather(s_v, [iv])
      own = jnp.full((16,), base + r, jnp.int32)
      si = plsc.load_gather(s_v, [own])
      att = ev * _rsqrt16(si * nbr + 1e-10)
      ks, vs = plsc.sort_key_val(iv, att)
      c_inc = plsc.cumsum(vs)
      c_exc = c_inc - vs
      prev = _gather16(ks, jnp.maximum(iota - 1, 0))
      nxt = _gather16(ks, jnp.minimum(iota + 1, 15))
      first = (iota == 0) | (ks != prev)
      last = (iota == 15) | (ks != nxt)
      seg_base = plsc.cummax(jnp.where(first, c_exc, -3e38))
      w = c_inc - seg_base
      plsc.addupdate_scatter(rowg_v, [ks + g * JP], w, mask=last)
      return c2
    lax.fori_loop(0, G * (KP // 16), _build, 0)

    pltpu.sync_copy(rowg_v, m_hbm.at[pl.ds((base + row0) * JP, G * JP)])

    def _clean(t, c2):
      g = t // (KP // 16)
      ck = t % (KP // 16)
      off = (row0 + g) * KP + ck * 16
      iv = idx_v[pl.ds(off, 16)]
      plsc.store_scatter(rowg_v, [iv + g * JP], zero16f)
      return c2
    lax.fori_loop(0, G * (KP // 16), _clean, 0)
    return c
  lax.fori_loop(0, NB, _batch, 0)


def _sc_densify(idx_flat, ew_flat, s_pad):
  mesh = plsc.VectorSubcoreMesh(
      core_axis_name="c", subcore_axis_name="s", num_cores=NC, num_subcores=NS)
  f = pl.kernel(
      _densify_body,
      out_type=jax.ShapeDtypeStruct((IP * JP,), jnp.float32),
      mesh=mesh,
      compiler_params=pltpu.CompilerParams(needs_layout_passes=False),
      scratch_types=[
          pltpu.VMEM((IP,), jnp.float32),
          pltpu.VMEM((RPT * KP,), jnp.int32),
          pltpu.VMEM((RPT * KP,), jnp.float32),
          pltpu.VMEM((G * JP,), jnp.float32),
      ],
  )
  return f(idx_flat, ew_flat, s_pad)


def _gather_body(u_ref, adj_ref, out_ref):
  out_ref[:, :, :I] = adj_ref[...]
  out_ref[:, :, I:] = jnp.zeros((1, 1, JP - I), jnp.float32)


def _gather_rows(users, adj_matrix):
  grid_spec = pltpu.PrefetchScalarGridSpec(
      num_scalar_prefetch=1,
      grid=(B,),
      in_specs=[pl.BlockSpec((1, 1, I), lambda b, u: (u[b], 0, 0))],
      out_specs=pl.BlockSpec((1, 1, JP), lambda b, u: (b, 0, 0)),
  )
  out = pl.pallas_call(
      _gather_body,
      grid_spec=grid_spec,
      out_shape=jax.ShapeDtypeStruct((B, 1, JP), jnp.float32),
  )(users, adj_matrix.reshape(10000, 1, I))
  return out.reshape(B, JP)


IT = 512
JT = 1024


def _mm_body(up_ref, m_ref, o_ref):
  @pl.when(pl.program_id(1) == 0)
  def _():
    o_ref[...] = jnp.zeros_like(o_ref)
  o_ref[...] += lax.dot_general(
      up_ref[...], m_ref[...], (((1,), (1,)), ((), ())),
      preferred_element_type=jnp.float32)


def _matmul(up, m):
  return pl.pallas_call(
      _mm_body,
      grid=(IP // IT, JP // JT),
      in_specs=[
          pl.BlockSpec((B, JT), lambda i, j: (0, j)),
          pl.BlockSpec((IT, JT), lambda i, j: (i, j)),
      ],
      out_specs=pl.BlockSpec((B, IT), lambda i, j: (0, i)),
      out_shape=jax.ShapeDtypeStruct((B, IP), jnp.float32),
      compiler_params=pltpu.CompilerParams(
          dimension_semantics=("arbitrary", "arbitrary")),
  )(up, m)


def kernel(users, adj_matrix, top_k_indices, top_k_values, temperature):
  idx = top_k_indices.astype(jnp.int32)
  exp_w, sums = _attn_stats(top_k_values, temperature)
  idx_p = jnp.pad(idx, ((0, IP - I), (0, KP - K)))
  ew_p = jnp.pad(exp_w, ((0, IP - I), (0, KP - K)))
  s_p = jnp.pad(sums.reshape(-1), (0, IP - I), constant_values=1.0)
  m_flat = _sc_densify(idx_p.reshape(-1), ew_p.reshape(-1), s_p)
  m = m_flat.reshape(IP, JP)
  up = _gather_rows(users.astype(jnp.int32), adj_matrix)
  scores_p = _matmul(up, m)
  return scores_p[:, :I]


# speedup vs baseline: 4.6263x; 1.5522x over previous
import functools

import jax
import jax.numpy as jnp
from jax import lax
from jax.experimental import pallas as pl
from jax.experimental.pallas import tpu as pltpu
from jax.experimental.pallas import tpu_sc as plsc

I = 5000
K = 50
B = 1024
NC = 2
NS = 16
NW = NC * NS
IP = 5120
JP = 5120
KP = 64
RPT = IP // NW
G = 8
NB = RPT // G


def _rsqrt16(x):
  i = plsc.bitcast(x, jnp.int32)
  i = jnp.int32(0x5F3759DF) - lax.shift_right_logical(i, 1)
  y = plsc.bitcast(i, jnp.float32)
  hx = 0.5 * x
  for _ in range(3):
    y = y * (1.5 - hx * y * y)
  return y


def _attn_stats_body(t_ref, v_ref, ew_ref, s_ref):
  ew = jnp.exp(v_ref[...] / t_ref[0])
  ew_ref[...] = ew
  s_ref[...] = jnp.sum(ew, axis=1, keepdims=True)


def _attn_stats(top_k_values, temperature):
  return pl.pallas_call(
      _attn_stats_body,
      in_specs=[
          pl.BlockSpec(memory_space=pltpu.SMEM),
          pl.BlockSpec((I, K), lambda: (0, 0)),
      ],
      out_specs=[
          pl.BlockSpec((I, K), lambda: (0, 0)),
          pl.BlockSpec((I, 1), lambda: (0, 0)),
      ],
      out_shape=[
          jax.ShapeDtypeStruct((I, K), jnp.float32),
          jax.ShapeDtypeStruct((I, 1), jnp.float32),
      ],
  )(temperature, top_k_values)


_IOTA16 = None


def _gather16(x, pos):
  dnums = lax.GatherDimensionNumbers(
      offset_dims=(), collapsed_slice_dims=(0,), start_index_map=(0,))
  return lax.gather(x, pos[:, None], dnums, (1,),
                    mode=lax.GatherScatterMode.PROMISE_IN_BOUNDS)


def _densify_body(idx_hbm, ew_hbm, s_hbm, m_hbm, s_v, idx_v, ew_v, rowg_v):
  cid = lax.axis_index("c")
  sid = lax.axis_index("s")
  wid = sid * NC + cid
  base = wid * RPT

  pltpu.sync_copy(s_hbm, s_v)
  pltpu.sync_copy(idx_hbm.at[pl.ds(base * KP, RPT * KP)], idx_v)
  pltpu.sync_copy(ew_hbm.at[pl.ds(base * KP, RPT * KP)], ew_v)

  zero16f = jnp.zeros((16,), jnp.float32)
  iota = lax.iota(jnp.int32, 16)

  def _zrow(t, c):
    rowg_v[pl.ds(t * 16, 16)] = zero16f
    return c
  lax.fori_loop(0, G * JP // 16, _zrow, 0)

  def _batch(b, c):
    row0 = b * G

    def _build(t, c2):
      g = t // (KP // 16)
      ck = t % (KP // 16)
      r = row0 + g
      off = r * KP + ck * 16
      iv = idx_v[pl.ds(off, 16)]
      ev = ew_v[pl.ds(off, 16)]
      nbr = plsc.load_gather(s_v, [iv])
      own = jnp.full((16,), base + r, jnp.int32)
      si = plsc.load_gather(s_v, [own])
      att = ev * _rsqrt16(si * nbr + 1e-10)
      ks, vs = plsc.sort_key_val(iv, att)
      c_inc = plsc.cumsum(vs)
      c_exc = c_inc - vs
      prev = _gather16(ks, jnp.maximum(iota - 1, 0))
      nxt = _gather16(ks, jnp.minimum(iota + 1, 15))
      first = (iota == 0) | (ks != prev)
      last = (iota == 15) | (ks != nxt)
      seg_base = plsc.cummax(jnp.where(first, c_exc, -3e38))
      w = c_inc - seg_base
      plsc.addupdate_scatter(rowg_v, [ks + g * JP], w, mask=last)
      return c2
    lax.fori_loop(0, G * (KP // 16), _build, 0)

    pltpu.sync_copy(rowg_v, m_hbm.at[pl.ds((base + row0) * JP, G * JP)])

    def _clean(t, c2):
      g = t // (KP // 16)
      ck = t % (KP // 16)
      off = (row0 + g) * KP + ck * 16
      iv = idx_v[pl.ds(off, 16)]
      plsc.store_scatter(rowg_v, [iv + g * JP], zero16f)
      return c2
    lax.fori_loop(0, G * (KP // 16), _clean, 0)
    return c
  lax.fori_loop(0, NB, _batch, 0)


def _sc_densify(idx_flat, ew_flat, s_pad):
  mesh = plsc.VectorSubcoreMesh(
      core_axis_name="c", subcore_axis_name="s", num_cores=NC, num_subcores=NS)
  f = pl.kernel(
      _densify_body,
      out_type=jax.ShapeDtypeStruct((IP * JP,), jnp.float32),
      mesh=mesh,
      compiler_params=pltpu.CompilerParams(needs_layout_passes=False),
      scratch_types=[
          pltpu.VMEM((IP,), jnp.float32),
          pltpu.VMEM((RPT * KP,), jnp.int32),
          pltpu.VMEM((RPT * KP,), jnp.float32),
          pltpu.VMEM((G * JP,), jnp.float32),
      ],
  )
  return f(idx_flat, ew_flat, s_pad)


RB = 16


def _gather_body(u_ref, *refs):
  out_ref = refs[RB]
  zero_tail = jnp.zeros((1, JP - I), jnp.float32)
  for r in range(RB):
    out_ref[r, :, :I] = refs[r][0]
    out_ref[r, :, I:] = zero_tail


def _gather_rows(users, adj_matrix):
  def mk_spec(r):
    return pl.BlockSpec((1, 1, I), lambda b, u, r=r: (u[b * RB + r], 0, 0))
  grid_spec = pltpu.PrefetchScalarGridSpec(
      num_scalar_prefetch=1,
      grid=(B // RB,),
      in_specs=[mk_spec(r) for r in range(RB)],
      out_specs=pl.BlockSpec((RB, 1, JP), lambda b, u: (b, 0, 0)),
  )
  out = pl.pallas_call(
      _gather_body,
      grid_spec=grid_spec,
      out_shape=jax.ShapeDtypeStruct((B, 1, JP), jnp.float32),
  )(users, *([adj_matrix.reshape(10000, 1, I)] * RB))
  return out.reshape(B, JP)


IT = 512
JT = 1024


def _mm_body(up_ref, m_ref, o_ref):
  @pl.when(pl.program_id(1) == 0)
  def _():
    o_ref[...] = jnp.zeros_like(o_ref)
  o_ref[...] += lax.dot_general(
      up_ref[...], m_ref[...], (((1,), (1,)), ((), ())),
      preferred_element_type=jnp.float32)


def _matmul(up, m):
  return pl.pallas_call(
      _mm_body,
      grid=(IP // IT, JP // JT),
      in_specs=[
          pl.BlockSpec((B, JT), lambda i, j: (0, j)),
          pl.BlockSpec((IT, JT), lambda i, j: (i, j)),
      ],
      out_specs=pl.BlockSpec((B, IT), lambda i, j: (0, i)),
      out_shape=jax.ShapeDtypeStruct((B, IP), jnp.float32),
      compiler_params=pltpu.CompilerParams(
          dimension_semantics=("arbitrary", "arbitrary")),
  )(up, m)


def kernel(users, adj_matrix, top_k_indices, top_k_values, temperature):
  idx = top_k_indices.astype(jnp.int32)
  exp_w, sums = _attn_stats(top_k_values, temperature)
  idx_p = jnp.pad(idx, ((0, IP - I), (0, KP - K)))
  ew_p = jnp.pad(exp_w, ((0, IP - I), (0, KP - K)))
  s_p = jnp.pad(sums.reshape(-1), (0, IP - I), constant_values=1.0)
  m_flat = _sc_densify(idx_p.reshape(-1), ew_p.reshape(-1), s_p)
  m = m_flat.reshape(IP, JP)
  up = _gather_rows(users.astype(jnp.int32), adj_matrix)
  scores_p = _matmul(up, m)
  return scores_p[:, :I]


# speedup vs baseline: 6.1968x; 1.3395x over previous
import functools

import jax
import jax.numpy as jnp
from jax import lax
from jax.experimental import pallas as pl
from jax.experimental.pallas import tpu as pltpu
from jax.experimental.pallas import tpu_sc as plsc

I = 5000
K = 50
B = 1024
NC = 2
NS = 16
NW = NC * NS
IP = 5120
JP = 5120
KP = 64
RPT = IP // NW
G = 8
NB = RPT // G


def _rsqrt16(x):
  i = plsc.bitcast(x, jnp.int32)
  i = jnp.int32(0x5F3759DF) - lax.shift_right_logical(i, 1)
  y = plsc.bitcast(i, jnp.float32)
  hx = 0.5 * x
  for _ in range(3):
    y = y * (1.5 - hx * y * y)
  return y


def _attn_stats_body(t_ref, v_ref, ew_ref, s_ref):
  ew = jnp.exp(v_ref[...] / t_ref[0])
  ew_ref[...] = ew
  s_ref[...] = jnp.sum(ew, axis=1, keepdims=True)


def _attn_stats(top_k_values, temperature):
  return pl.pallas_call(
      _attn_stats_body,
      in_specs=[
          pl.BlockSpec(memory_space=pltpu.SMEM),
          pl.BlockSpec((I, K), lambda: (0, 0)),
      ],
      out_specs=[
          pl.BlockSpec((I, K), lambda: (0, 0)),
          pl.BlockSpec((I, 1), lambda: (0, 0)),
      ],
      out_shape=[
          jax.ShapeDtypeStruct((I, K), jnp.float32),
          jax.ShapeDtypeStruct((I, 1), jnp.float32),
      ],
  )(temperature, top_k_values)


_IOTA16 = None


def _gather16(x, pos):
  dnums = lax.GatherDimensionNumbers(
      offset_dims=(), collapsed_slice_dims=(0,), start_index_map=(0,))
  return lax.gather(x, pos[:, None], dnums, (1,),
                    mode=lax.GatherScatterMode.PROMISE_IN_BOUNDS)


def _densify_body(idx_hbm, ew_hbm, s_hbm, m_hbm, s_v, idx_v, ew_v, rowg_v):
  cid = lax.axis_index("c")
  sid = lax.axis_index("s")
  wid = sid * NC + cid
  base = wid * RPT

  pltpu.sync_copy(s_hbm, s_v)
  pltpu.sync_copy(idx_hbm.at[pl.ds(base * KP, RPT * KP)], idx_v)
  pltpu.sync_copy(ew_hbm.at[pl.ds(base * KP, RPT * KP)], ew_v)

  zero16f = jnp.zeros((16,), jnp.float32)
  iota = lax.iota(jnp.int32, 16)

  def _zrow(t, c):
    rowg_v[pl.ds(t * 16, 16)] = zero16f
    return c
  lax.fori_loop(0, G * JP // 16, _zrow, 0)

  def _batch(b, c):
    row0 = b * G

    def _build(t, c2):
      g = t // (KP // 16)
      ck = t % (KP // 16)
      r = row0 + g
      off = r * KP + ck * 16
      iv = idx_v[pl.ds(off, 16)]
      ev = ew_v[pl.ds(off, 16)]
      nbr = plsc.load_gather(s_v, [iv])
      own = jnp.full((16,), base + r, jnp.int32)
      si = plsc.load_gather(s_v, [own])
      att = ev * _rsqrt16(si * nbr + 1e-10)
      ks, vs = plsc.sort_key_val(iv, att)
      c_inc = plsc.cumsum(vs)
      c_exc = c_inc - vs
      prev = _gather16(ks, jnp.maximum(iota - 1, 0))
      nxt = _gather16(ks, jnp.minimum(iota + 1, 15))
      first = (iota == 0) | (ks != prev)
      last = (iota == 15) | (ks != nxt)
      seg_base = plsc.cummax(jnp.where(first, c_exc, -3e38))
      w = c_inc - seg_base
      plsc.addupdate_scatter(rowg_v, [ks + g * JP], w, mask=last)
      return c2
    lax.fori_loop(0, G * (KP // 16), _build, 0)

    pltpu.sync_copy(rowg_v, m_hbm.at[pl.ds((base + row0) * JP, G * JP)])

    def _clean(t, c2):
      g = t // (KP // 16)
      ck = t % (KP // 16)
      off = (row0 + g) * KP + ck * 16
      iv = idx_v[pl.ds(off, 16)]
      plsc.store_scatter(rowg_v, [iv + g * JP], zero16f)
      return c2
    lax.fori_loop(0, G * (KP // 16), _clean, 0)
    return c
  lax.fori_loop(0, NB, _batch, 0)


def _sc_densify(idx_flat, ew_flat, s_pad):
  mesh = plsc.VectorSubcoreMesh(
      core_axis_name="c", subcore_axis_name="s", num_cores=NC, num_subcores=NS)
  f = pl.kernel(
      _densify_body,
      out_type=jax.ShapeDtypeStruct((IP * JP,), jnp.float32),
      mesh=mesh,
      compiler_params=pltpu.CompilerParams(needs_layout_passes=False),
      scratch_types=[
          pltpu.VMEM((IP,), jnp.float32),
          pltpu.VMEM((RPT * KP,), jnp.int32),
          pltpu.VMEM((RPT * KP,), jnp.float32),
          pltpu.VMEM((G * JP,), jnp.float32),
      ],
  )
  return f(idx_flat, ew_flat, s_pad)


U = 10000
UT = 2000
JT2 = 1024


def _sel_body(u_ref, adj_ref, o_ref):
  u = pl.program_id(1)
  nu = pl.num_programs(1)
  j2 = pl.program_id(0)

  @pl.when(u == 0)
  def _():
    o_ref[...] = jnp.zeros_like(o_ref)
  col = lax.broadcasted_iota(jnp.int32, (B, UT), 1) + u * UT
  oh = (u_ref[...] == col).astype(jnp.bfloat16)
  o_ref[...] += lax.dot_general(
      oh, adj_ref[...].astype(jnp.bfloat16), (((1,), (0,)), ((), ())),
      preferred_element_type=jnp.float32)

  @pl.when((u == nu - 1) & (j2 == JP // JT2 - 1))
  def _():
    cmask = lax.broadcasted_iota(jnp.int32, (B, JT2), 1) < (I - 4 * JT2)
    o_ref[...] = jnp.where(cmask, o_ref[...], 0.0)


def _select_rows(users, adj_matrix):
  return pl.pallas_call(
      _sel_body,
      grid=(JP // JT2, U // UT),
      in_specs=[
          pl.BlockSpec((B, 1), lambda j2, u: (0, 0)),
          pl.BlockSpec((UT, JT2), lambda j2, u: (u, j2)),
      ],
      out_specs=pl.BlockSpec((B, JT2), lambda j2, u: (0, j2)),
      out_shape=jax.ShapeDtypeStruct((B, JP), jnp.float32),
      compiler_params=pltpu.CompilerParams(
          dimension_semantics=("arbitrary", "arbitrary")),
  )(users.reshape(B, 1), adj_matrix)


IT = 512
JT = 1024


def _mm_body(up_ref, m_ref, o_ref):
  @pl.when(pl.program_id(1) == 0)
  def _():
    o_ref[...] = jnp.zeros_like(o_ref)
  o_ref[...] += lax.dot_general(
      up_ref[...].astype(jnp.bfloat16), m_ref[...].astype(jnp.bfloat16),
      (((1,), (1,)), ((), ())),
      preferred_element_type=jnp.float32)


def _matmul(up, m):
  return pl.pallas_call(
      _mm_body,
      grid=(IP // IT, JP // JT),
      in_specs=[
          pl.BlockSpec((B, JT), lambda i, j: (0, j)),
          pl.BlockSpec((IT, JT), lambda i, j: (i, j)),
      ],
      out_specs=pl.BlockSpec((B, IT), lambda i, j: (0, i)),
      out_shape=jax.ShapeDtypeStruct((B, IP), jnp.float32),
      compiler_params=pltpu.CompilerParams(
          dimension_semantics=("arbitrary", "arbitrary")),
  )(up, m)


def kernel(users, adj_matrix, top_k_indices, top_k_values, temperature):
  idx = top_k_indices.astype(jnp.int32)
  exp_w, sums = _attn_stats(top_k_values, temperature)
  idx_p = jnp.pad(idx, ((0, IP - I), (0, KP - K)))
  ew_p = jnp.pad(exp_w, ((0, IP - I), (0, KP - K)))
  s_p = jnp.pad(sums.reshape(-1), (0, IP - I), constant_values=1.0)
  m_flat = _sc_densify(idx_p.reshape(-1), ew_p.reshape(-1), s_p)
  m = m_flat.reshape(IP, JP)
  up = _select_rows(users.astype(jnp.int32), adj_matrix)
  scores_p = _matmul(up, m)
  return scores_p[:, :I]
